# baseline (speedup 1.0000x reference)
import functools

import jax
import jax.numpy as jnp
from jax import lax
from jax.experimental import pallas as pl
from jax.experimental.pallas import tpu as pltpu
from jax.experimental.pallas import tpu_sc as plsc

_N = 8192
_EPS = 1e-5
_F32 = jnp.float32
_BRL = 4096


def _leaky(x):
    return jnp.where(x >= 0, x, 0.2 * x)


def _dot(a, b):
    return jax.lax.dot(a, b, preferred_element_type=_F32)


def _norm_apply(y, m, v, g, be):
    brl, c = y.shape
    G = m.shape[1]
    cg = c // G
    if G == 1:
        mc, vc = m, v
    else:
        mc = jnp.concatenate(
            [jnp.broadcast_to(m[:, i:i + 1], (brl, cg)) for i in range(G)], 1)
        vc = jnp.concatenate(
            [jnp.broadcast_to(v[:, i:i + 1], (brl, cg)) for i in range(G)], 1)
    return (y - mc) / jnp.sqrt(vc + _EPS) * g + be


def _stats(y, G):
    n, c = y.shape
    xg = y.reshape(n, G, c // G)
    return xg.mean(axis=2), xg.var(axis=2)


def _lin_body(x_ref, w_ref, b_ref, o_ref):
    o_ref[...] = _leaky(_dot(x_ref[...], w_ref[...]) + b_ref[...])


def _run_lin1(x, wt, b):
    m, k = x.shape
    n = wt.shape[1]
    return pl.pallas_call(
        _lin_body,
        grid=(m // _BRL,),
        in_specs=[
            pl.BlockSpec((_BRL, k), lambda r: (r, 0)),
            pl.BlockSpec((k, n), lambda r: (0, 0)),
            pl.BlockSpec((1, n), lambda r: (0, 0)),
        ],
        out_specs=pl.BlockSpec((_BRL, n), lambda r: (r, 0)),
        out_shape=jax.ShapeDtypeStruct((m, n), _F32),
        compiler_params=pltpu.CompilerParams(
            dimension_semantics=("arbitrary",)),
    )(x, wt, b)


def _gnlin_body(y_ref, m_ref, v_ref, g_ref, e_ref, w_ref, b_ref, o_ref,
                *, act, dual):
    a = _norm_apply(y_ref[...], m_ref[...], v_ref[...], g_ref[...], e_ref[...])
    z = _dot(a, w_ref[...]) + b_ref[...]
    if act:
        z = _leaky(z)
    if dual:
        o_ref[0][...] = a
        o_ref[1][...] = z
    else:
        o_ref[...] = z


def _run_gnlin(y, m, v, g, be, wt, b, act=True, dual=False):
    rows, k = y.shape
    n = wt.shape[1]
    G = m.shape[1]
    r2 = lambda a: a.reshape(1, -1)
    body = functools.partial(_gnlin_body, act=act, dual=dual)

    def wrapped(y_ref, m_ref, v_ref, g_ref, e_ref, w_ref, b_ref, *outs):
        body(y_ref, m_ref, v_ref, g_ref, e_ref, w_ref, b_ref,
             outs if dual else outs[0])

    out_shape = [jax.ShapeDtypeStruct((rows, k), _F32),
                 jax.ShapeDtypeStruct((rows, n), _F32)] if dual else \
        jax.ShapeDtypeStruct((rows, n), _F32)
    out_specs = [pl.BlockSpec((_BRL, k), lambda r: (r, 0)),
                 pl.BlockSpec((_BRL, n), lambda r: (r, 0))] if dual else \
        pl.BlockSpec((_BRL, n), lambda r: (r, 0))
    return pl.pallas_call(
        wrapped,
        grid=(rows // _BRL,),
        in_specs=[
            pl.BlockSpec((_BRL, k), lambda r: (r, 0)),
            pl.BlockSpec((_BRL, G), lambda r: (r, 0)),
            pl.BlockSpec((_BRL, G), lambda r: (r, 0)),
            pl.BlockSpec((1, k), lambda r: (0, 0)),
            pl.BlockSpec((1, k), lambda r: (0, 0)),
            pl.BlockSpec((k, n), lambda r: (0, 0)),
            pl.BlockSpec((1, n), lambda r: (0, 0)),
        ],
        out_specs=out_specs,
        out_shape=out_shape,
        compiler_params=pltpu.CompilerParams(
            dimension_semantics=("arbitrary",)),
    )(y, m, v, r2(g), r2(be), wt, r2(b))


def _cat5_body(x1_ref, gm_ref, w_ref, b_ref, o_ref):
    gb = jnp.broadcast_to(gm_ref[0][:1], (_BRL, 128))
    cat = jnp.concatenate([gb, x1_ref[0]], axis=1)
    o_ref[0] = _leaky(_dot(cat, w_ref[...]) + b_ref[...])


def _run_cat5(x1v, gm8, wt, b):
    return pl.pallas_call(
        _cat5_body,
        grid=(2, _N // _BRL),
        in_specs=[
            pl.BlockSpec((1, _BRL, 128), lambda c, r: (c, r, 0)),
            pl.BlockSpec((1, 8, 128), lambda c, r: (c, 0, 0)),
            pl.BlockSpec((256, 128), lambda c, r: (0, 0)),
            pl.BlockSpec((1, 128), lambda c, r: (0, 0)),
        ],
        out_specs=pl.BlockSpec((1, _BRL, 128), lambda c, r: (c, r, 0)),
        out_shape=jax.ShapeDtypeStruct((2, _N, 128), _F32),
        compiler_params=pltpu.CompilerParams(
            dimension_semantics=("arbitrary", "arbitrary")),
    )(x1v, gm8, wt, b.reshape(1, -1))


def _catf_body(ovn_ref, corr_ref, mx_ref, w_ref, b_ref, o_ref):
    cat = jnp.concatenate(
        [ovn_ref[...], corr_ref[...], mx_ref[...],
         jnp.zeros((_BRL, 7), _F32)], axis=1)
    o_ref[...] = _leaky(_dot(cat, w_ref[...]) + b_ref[...])


def _run_catf(ovn, corr, mx, wft, bf):
    return pl.pallas_call(
        _catf_body,
        grid=(_N // _BRL,),
        in_specs=[
            pl.BlockSpec((_BRL, 128), lambda r: (r, 0)),
            pl.BlockSpec((_BRL, 128), lambda r: (r, 0)),
            pl.BlockSpec((_BRL, 1), lambda r: (r, 0)),
            pl.BlockSpec((264, 64), lambda r: (0, 0)),
            pl.BlockSpec((1, 64), lambda r: (0, 0)),
        ],
        out_specs=pl.BlockSpec((_BRL, 64), lambda r: (r, 0)),
        out_shape=jax.ShapeDtypeStruct((_N, 64), _F32),
        compiler_params=pltpu.CompilerParams(
            dimension_semantics=("arbitrary",)),
    )(ovn, corr, mx, wft, bf.reshape(1, -1))


_RBS = 1024
_CBS = 1024


def _sim_body(ovn_ref, opnt_ref, idx_ref, mx_ref, m_sc, i_sc):
    cb = pl.program_id(1)
    ncb = pl.num_programs(1)
    s = _dot(ovn_ref[...], opnt_ref[...])
    bm = jnp.max(s, axis=1, keepdims=True)
    col = lax.broadcasted_iota(jnp.int32, s.shape, 1) + cb * _CBS
    cand = jnp.min(jnp.where(s == bm, col, jnp.int32(2 ** 30)),
                   axis=1, keepdims=True)

    @pl.when(cb == 0)
    def _():
        m_sc[...] = bm
        i_sc[...] = cand

    @pl.when(cb > 0)
    def _():
        prev = m_sc[...]
        better = bm > prev
        i_sc[...] = jnp.where(better, cand, i_sc[...])
        m_sc[...] = jnp.where(better, bm, prev)

    @pl.when(cb == ncb - 1)
    def _():
        idx_ref[...] = i_sc[...]
        mx_ref[...] = m_sc[...]


def _run_sim_argmax(ovn, opn_t):
    grid = (_N // _RBS, _N // _CBS)
    return pl.pallas_call(
        _sim_body,
        grid=grid,
        in_specs=[
            pl.BlockSpec((_RBS, 128), lambda rb, cb: (rb, 0)),
            pl.BlockSpec((128, _CBS), lambda rb, cb: (0, cb)),
        ],
        out_specs=[
            pl.BlockSpec((_RBS, 1), lambda rb, cb: (rb, 0)),
            pl.BlockSpec((_RBS, 1), lambda rb, cb: (rb, 0)),
        ],
        out_shape=[
            jax.ShapeDtypeStruct((_N, 1), jnp.int32),
            jax.ShapeDtypeStruct((_N, 1), _F32),
        ],
        scratch_shapes=[
            pltpu.VMEM((_RBS, 1), _F32),
            pltpu.VMEM((_RBS, 1), jnp.int32),
        ],
        compiler_params=pltpu.CompilerParams(
            dimension_semantics=("parallel", "arbitrary")),
    )(ovn, opn_t)


def _make_sc_gather():
    info = plsc.get_sparse_core_info()
    nc, ns = info.num_cores, info.num_subcores
    nw = nc * ns
    bpw = _N // nw
    mesh = plsc.VectorSubcoreMesh(core_axis_name="c", subcore_axis_name="s")

    @functools.partial(
        pl.kernel,
        mesh=mesh,
        out_type=jax.ShapeDtypeStruct((_N, 128), _F32),
        scratch_types=[
            pltpu.VMEM((bpw,), jnp.int32),
            pltpu.VMEM((bpw, 128), _F32),
            pltpu.SemaphoreType.DMA,
        ],
    )
    def gather(table_hbm, idx_hbm, out_hbm, idx_v, rows_v, sem):
        wid = lax.axis_index("s") * nc + lax.axis_index("c")
        base = wid * bpw
        pltpu.sync_copy(idx_hbm.at[pl.ds(base, bpw)], idx_v)
        pltpu.async_copy(table_hbm.at[idx_v], rows_v, sem).wait()
        pltpu.sync_copy(rows_v, out_hbm.at[pl.ds(base, bpw)])

    return gather


def kernel(vtx, pts, params):
    p = params
    n2 = 2 * _N

    x = jnp.zeros((n2, 8), _F32)
    x = x.at[:_N, :3].set(vtx).at[_N:, :3].set(pts)
    w1t = jnp.zeros((8, 32), _F32).at[:3].set(p['W1'].T)

    y1 = _run_lin1(x, w1t, p['b1'].reshape(1, -1))
    m1, v1 = _stats(y1, 1)
    y2 = _run_gnlin(y1, m1, v1, p['g1'], p['be1'], p['W2'].T, p['b2'])
    m2, v2 = _stats(y2, 2)
    y3 = _run_gnlin(y2, m2, v2, p['g2'], p['be2'], p['W3'].T, p['b3'])
    m3, v3 = _stats(y3, 4)
    x1, y4 = _run_gnlin(y3, m3, v3, p['g3'], p['be3'], p['W4'].T, p['b4'],
                        dual=True)
    m4, v4 = _stats(y4, 4)
    x2 = _norm_apply(y4, m4, v4, p['g4'].reshape(1, -1),
                     p['be4'].reshape(1, -1))
    gmax_v = jnp.max(x2[:_N], axis=0)
    gmax_p = jnp.max(x2[_N:], axis=0)
    gm8 = jnp.broadcast_to(
        jnp.stack([gmax_v, gmax_p])[:, None, :], (2, 8, 128))

    y5 = _run_cat5(x1.reshape(2, _N, 128), gm8, p['W5'].T, p['b5'])
    y5 = y5.reshape(n2, 128)
    m5, v5 = _stats(y5, 4)
    y6 = _run_gnlin(y5, m5, v5, p['g5'], p['be5'], p['W6'].T, p['b6'])
    m6, v6 = _stats(y6, 2)
    o = _run_gnlin(y6, m6, v6, p['g6'], p['be6'], p['W7'].T, p['b7'],
                   act=False)
    on = o / jnp.linalg.norm(o, axis=1, keepdims=True)
    ovn, opn = on[:_N], on[_N:]

    idx2d, mx2d = _run_sim_argmax(ovn, opn.T)

    corr = _make_sc_gather()(opn, idx2d.reshape(_N))

    wft = jnp.zeros((264, 64), _F32).at[:257].set(p['Wf'].T)
    hf = _run_catf(ovn, corr, mx2d, wft, p['bf'])
    mf, vf = _stats(hf, 2)
    out_corrmask = _run_gnlin(hf, mf, vf, p['gf'], p['bef'], p['Wl'].T,
                              p['bl'], act=False)

    return ovn, opn, out_corrmask


# speedup vs baseline: 1.2596x; 1.2596x over previous
import functools

import jax
import jax.numpy as jnp
from jax import lax
from jax.experimental import pallas as pl
from jax.experimental.pallas import tpu as pltpu
from jax.experimental.pallas import tpu_sc as plsc

_N = 8192
_EPS = 1e-5
_F32 = jnp.float32


def _leaky(x):
    return jnp.where(x >= 0, x, 0.2 * x)


def _dot(a, b):
    return jax.lax.dot(a, b, preferred_element_type=_F32)


def _lin_leaky_body(x_ref, w_ref, b_ref, o_ref):
    o_ref[...] = _leaky(_dot(x_ref[...], w_ref[...]) + b_ref[...])


def _lin_body(x_ref, w_ref, b_ref, o_ref):
    o_ref[...] = _dot(x_ref[...], w_ref[...]) + b_ref[...]


def _run_lin(x, wt, b, act, brl=8192):
    m, k = x.shape
    n = wt.shape[1]
    body = _lin_leaky_body if act else _lin_body
    return pl.pallas_call(
        body,
        grid=(m // brl,),
        in_specs=[
            pl.BlockSpec((brl, k), lambda r: (r, 0)),
            pl.BlockSpec((k, n), lambda r: (0, 0)),
            pl.BlockSpec((1, n), lambda r: (0, 0)),
        ],
        out_specs=pl.BlockSpec((brl, n), lambda r: (r, 0)),
        out_shape=jax.ShapeDtypeStruct((m, n), _F32),
        compiler_params=pltpu.CompilerParams(
            dimension_semantics=("arbitrary",)),
    )(x, wt, b)


def _gn(y, G, w, b):
    n, c = y.shape
    xg = y.reshape(n, G, c // G)
    m = xg.mean(axis=2, keepdims=True)
    v = xg.var(axis=2, keepdims=True)
    xn = (xg - m) / jnp.sqrt(v + _EPS)
    return xn.reshape(n, c) * w + b


_RBS = 1024
_CBS = 1024


def _sim_body(ovn_ref, opnt_ref, idx_ref, mx_ref, m_sc, i_sc):
    cb = pl.program_id(1)
    ncb = pl.num_programs(1)
    s = _dot(ovn_ref[...], opnt_ref[...])
    bm = jnp.max(s, axis=1, keepdims=True)
    col = lax.broadcasted_iota(jnp.int32, s.shape, 1) + cb * _CBS
    cand = jnp.min(jnp.where(s == bm, col, jnp.int32(2 ** 30)),
                   axis=1, keepdims=True)

    @pl.when(cb == 0)
    def _():
        m_sc[...] = bm
        i_sc[...] = cand

    @pl.when(cb > 0)
    def _():
        prev = m_sc[...]
        better = bm > prev
        i_sc[...] = jnp.where(better, cand, i_sc[...])
        m_sc[...] = jnp.where(better, bm, prev)

    @pl.when(cb == ncb - 1)
    def _():
        idx_ref[...] = i_sc[...]
        mx_ref[...] = m_sc[...]


def _run_sim_argmax(ovn, opn_t):
    grid = (_N // _RBS, _N // _CBS)
    return pl.pallas_call(
        _sim_body,
        grid=grid,
        in_specs=[
            pl.BlockSpec((_RBS, 128), lambda rb, cb: (rb, 0)),
            pl.BlockSpec((128, _CBS), lambda rb, cb: (0, cb)),
        ],
        out_specs=[
            pl.BlockSpec((_RBS, 1), lambda rb, cb: (rb, 0)),
            pl.BlockSpec((_RBS, 1), lambda rb, cb: (rb, 0)),
        ],
        out_shape=[
            jax.ShapeDtypeStruct((_N, 1), jnp.int32),
            jax.ShapeDtypeStruct((_N, 1), _F32),
        ],
        scratch_shapes=[
            pltpu.VMEM((_RBS, 1), _F32),
            pltpu.VMEM((_RBS, 1), jnp.int32),
        ],
        compiler_params=pltpu.CompilerParams(
            dimension_semantics=("parallel", "arbitrary")),
    )(ovn, opn_t)


def _make_sc_gather():
    info = plsc.get_sparse_core_info()
    nc, ns = info.num_cores, info.num_subcores
    nw = nc * ns
    bpw = _N // nw
    mesh = plsc.VectorSubcoreMesh(core_axis_name="c", subcore_axis_name="s")

    @functools.partial(
        pl.kernel,
        mesh=mesh,
        out_type=jax.ShapeDtypeStruct((_N, 128), _F32),
        scratch_types=[
            pltpu.VMEM((bpw,), jnp.int32),
            pltpu.VMEM((bpw, 128), _F32),
            pltpu.SemaphoreType.DMA,
        ],
    )
    def gather(table_hbm, idx_hbm, out_hbm, idx_v, rows_v, sem):
        wid = lax.axis_index("s") * nc + lax.axis_index("c")
        base = wid * bpw
        pltpu.sync_copy(idx_hbm.at[pl.ds(base, bpw)], idx_v)
        pltpu.async_copy(table_hbm.at[idx_v], rows_v, sem).wait()
        pltpu.sync_copy(rows_v, out_hbm.at[pl.ds(base, bpw)])

    return gather


def kernel(vtx, pts, params):
    p = params
    r2 = lambda a: a.reshape(1, -1)
    n2 = 2 * _N

    x = jnp.zeros((n2, 8), _F32)
    x = x.at[:_N, :3].set(vtx).at[_N:, :3].set(pts)
    w1t = jnp.zeros((8, 32), _F32).at[:3].set(p['W1'].T)

    y1 = _run_lin(x, w1t, r2(p['b1']), True)
    a1 = _gn(y1, 1, p['g1'], p['be1'])
    y2 = _run_lin(a1, p['W2'].T, r2(p['b2']), True)
    a2 = _gn(y2, 2, p['g2'], p['be2'])
    y3 = _run_lin(a2, p['W3'].T, r2(p['b3']), True)
    x1 = _gn(y3, 4, p['g3'], p['be3'])
    y4 = _run_lin(x1, p['W4'].T, r2(p['b4']), True)
    x2 = _gn(y4, 4, p['g4'], p['be4'])

    gmax_v = jnp.max(x2[:_N], axis=0)
    gmax_p = jnp.max(x2[_N:], axis=0)
    cat5 = jnp.concatenate([
        jnp.concatenate([jnp.tile(gmax_v[None, :], (_N, 1)), x1[:_N]], axis=1),
        jnp.concatenate([jnp.tile(gmax_p[None, :], (_N, 1)), x1[_N:]], axis=1),
    ], axis=0)

    y5 = _run_lin(cat5, p['W5'].T, r2(p['b5']), True)
    a5 = _gn(y5, 4, p['g5'], p['be5'])
    y6 = _run_lin(a5, p['W6'].T, r2(p['b6']), True)
    a6 = _gn(y6, 2, p['g6'], p['be6'])
    o = _run_lin(a6, p['W7'].T, r2(p['b7']), False)
    on = o / jnp.linalg.norm(o, axis=1, keepdims=True)
    ovn, opn = on[:_N], on[_N:]

    idx2d, mx2d = _run_sim_argmax(ovn, opn.T)

    corr = _make_sc_gather()(opn, idx2d.reshape(_N))

    catf = jnp.concatenate(
        [ovn, corr, mx2d, jnp.zeros((_N, 7), _F32)], axis=1)
    wft = jnp.zeros((264, 64), _F32).at[:257].set(p['Wf'].T)
    hf = _run_lin(catf, wft, r2(p['bf']), True)
    af = _gn(hf, 2, p['gf'], p['bef'])
    out_corrmask = _run_lin(af, p['Wl'].T, r2(p['bl']), False)

    return ovn, opn, out_corrmask
